# Initial kernel scaffold; baseline (speedup 1.0000x reference)
import jax
import jax.numpy as jnp
from jax.experimental import pallas as pl


def kernel(x, edge_index, batch, Wg00, bg00, gam0, bet0, Wg01, bg01, Wg10, bg10, gam1, bet1, Wg11, bg11, Wg20, bg20, gam2, bet2, Wg21, bg21, Wp0, bp0, gamp, betp, Wp1, bp1):
    raise NotImplementedError("write your pallas kernel here")


# speedup vs baseline: 6.6325x; 6.6325x over previous
import functools

import jax
import jax.numpy as jnp
from jax import lax
from jax.experimental import pallas as pl
from jax.experimental.pallas import tpu as pltpu
from jax.experimental.pallas import tpu_sc as plsc

N = 10000
E = 320000
H = 128
G = 64
C = 10
EPS_BN = 1e-5

NC = 2
NS = 16
NW = NC * NS
EPW = E // NW
K = 80
CH = EPW // K
RPT = 624
TAIL = N - NS * RPT

_mesh = plsc.VectorSubcoreMesh(core_axis_name="c", subcore_axis_name="s")


@functools.partial(
    pl.kernel,
    out_type=jax.ShapeDtypeStruct((NC, N, H), jnp.float32),
    mesh=_mesh,
    scratch_types=[
        pltpu.VMEM((CH, K), jnp.int32),
        pltpu.VMEM((CH, K), jnp.int32),
        pltpu.VMEM((K, H), jnp.float32),
        pltpu.VMEM_SHARED((N, H), jnp.float32),
        pltpu.SemaphoreType.DMA,
    ],
)
def _sc_agg(h_hbm, src_hbm, dst_hbm, zeros_hbm, out_hbm,
            src_v, dst_v, rows_v, agg_sh, sem):
    c = lax.axis_index("c")
    s = lax.axis_index("s")
    r0 = s * RPT
    pltpu.sync_copy(zeros_hbm.at[pl.ds(r0, RPT)], agg_sh.at[pl.ds(r0, RPT)])

    @pl.when(s == 0)
    def _zero_tail():
        pltpu.sync_copy(zeros_hbm.at[pl.ds(NS * RPT, TAIL)],
                        agg_sh.at[pl.ds(NS * RPT, TAIL)])
    pltpu.sync_copy(src_hbm.at[c, s], src_v)
    pltpu.sync_copy(dst_hbm.at[c, s], dst_v)
    plsc.subcore_barrier()

    def body(i, carry):
        pltpu.async_copy(h_hbm.at[src_v.at[i]], rows_v, sem).wait()
        pltpu.sync_copy(rows_v, agg_sh.at[dst_v.at[i]], add=True)
        return carry

    lax.fori_loop(0, CH, body, 0)
    plsc.subcore_barrier()
    pltpu.sync_copy(agg_sh.at[pl.ds(r0, RPT)], out_hbm.at[c, pl.ds(r0, RPT)])

    @pl.when(s == 0)
    def _write_tail():
        pltpu.sync_copy(agg_sh.at[pl.ds(NS * RPT, TAIL)],
                        out_hbm.at[c, pl.ds(NS * RPT, TAIL)])


BN = 2000
NB = N // BN


def _mlp_body(h_ref, p_ref, w0_ref, b0_ref, w1_ref, b1_ref, out_ref):
    u = h_ref[...] + p_ref[0] + p_ref[1]
    y = jnp.dot(u, w0_ref[...], preferred_element_type=jnp.float32) + b0_ref[...]
    y = jnp.where(y >= 0.0, y, 0.1 * y)
    out_ref[...] = jnp.dot(y, w1_ref[...], preferred_element_type=jnp.float32) + b1_ref[...]


def _mlp(h, parts, w0t, b0, w1t, b1):
    return pl.pallas_call(
        _mlp_body,
        out_shape=jax.ShapeDtypeStruct((N, H), jnp.float32),
        grid=(NB,),
        in_specs=[
            pl.BlockSpec((BN, H), lambda i: (i, 0)),
            pl.BlockSpec((NC, BN, H), lambda i: (0, i, 0)),
            pl.BlockSpec((H, H), lambda i: (0, 0)),
            pl.BlockSpec((1, H), lambda i: (0, 0)),
            pl.BlockSpec((H, H), lambda i: (0, 0)),
            pl.BlockSpec((1, H), lambda i: (0, 0)),
        ],
        out_specs=pl.BlockSpec((BN, H), lambda i: (i, 0)),
    )(h, parts, w0t, b0, w1t, b1)


def _pool_body(b_ref, h_ref, w0_ref, b0_ref, w1_ref, b1_ref, out_ref, acc_ref):
    i = pl.program_id(0)

    @pl.when(i == 0)
    def _init():
        acc_ref[...] = jnp.zeros_like(acc_ref)

    seg = b_ref[0]
    onehot = (lax.broadcasted_iota(jnp.int32, (G, BN), 0) == seg).astype(jnp.float32)
    acc_ref[...] += jnp.dot(onehot, h_ref[...], preferred_element_type=jnp.float32)

    @pl.when(i == NB - 1)
    def _fin():
        y = jnp.dot(acc_ref[...], w0_ref[...], preferred_element_type=jnp.float32) + b0_ref[...]
        y = jnp.where(y >= 0.0, y, 0.1 * y)
        out_ref[...] = jnp.dot(y, w1_ref[...], preferred_element_type=jnp.float32) + b1_ref[...]


def _pool(batch3, h, w0t, b0, w1t, b1):
    return pl.pallas_call(
        _pool_body,
        out_shape=jax.ShapeDtypeStruct((G, H), jnp.float32),
        grid=(NB,),
        in_specs=[
            pl.BlockSpec((1, 1, BN), lambda i: (i, 0, 0)),
            pl.BlockSpec((BN, H), lambda i: (i, 0)),
            pl.BlockSpec((H, H), lambda i: (0, 0)),
            pl.BlockSpec((1, H), lambda i: (0, 0)),
            pl.BlockSpec((H, H), lambda i: (0, 0)),
            pl.BlockSpec((1, H), lambda i: (0, 0)),
        ],
        out_specs=pl.BlockSpec((G, H), lambda i: (0, 0)),
        scratch_shapes=[pltpu.VMEM((G, H), jnp.float32)],
    )(batch3, h, w0t, b0, w1t, b1)


def _fold_bn(W0, b0, gam, bet, W1, b1):
    scale = gam / jnp.sqrt(1.0 + EPS_BN)
    w0t = (W0 * scale[:, None]).T
    b0e = (b0 * scale + bet)[None, :]
    return w0t, b0e, W1.T, b1[None, :]


def kernel(x, edge_index, batch,
           Wg00, bg00, gam0, bet0, Wg01, bg01,
           Wg10, bg10, gam1, bet1, Wg11, bg11,
           Wg20, bg20, gam2, bet2, Wg21, bg21,
           Wp0, bp0, gamp, betp, Wp1, bp1):
    src = edge_index[0].reshape(NC, NS, CH, K)
    dst = edge_index[1].reshape(NC, NS, CH, K)
    zeros = jnp.zeros((N, H), jnp.float32)

    layers = [
        _fold_bn(Wg00, bg00, gam0, bet0, Wg01, bg01),
        _fold_bn(Wg10, bg10, gam1, bet1, Wg11, bg11),
        _fold_bn(Wg20, bg20, gam2, bet2, Wg21, bg21),
    ]

    h = x
    for w0t, b0e, w1t, b1e in layers:
        parts = _sc_agg(h, src, dst, zeros)
        h = _mlp(h, parts, w0t, b0e, w1t, b1e)

    batch3 = batch.reshape(NB, 1, BN)
    scalep = gamp / jnp.sqrt(1.0 + EPS_BN)
    wp0t = (Wp0 * scalep[:, None]).T
    bp0e = (bp0 * scalep + betp)[None, :]
    wp1t = jnp.zeros((H, H), jnp.float32).at[:, :C].set(Wp1.T)
    bp1e = jnp.zeros((1, H), jnp.float32).at[0, :C].set(bp1)

    y = _pool(batch3, h, wp0t, bp0e, wp1t, bp1e)
    return y[:, :C]
